# Initial kernel scaffold; baseline (speedup 1.0000x reference)
import jax
import jax.numpy as jnp
from jax.experimental import pallas as pl


def kernel(x_index, features_index, edge_index, edge_weight, embedding, W1, b1, W2, b2):
    raise NotImplementedError("write your pallas kernel here")


# speedup vs baseline: 1.8733x; 1.8733x over previous
import functools

import jax
import jax.numpy as jnp
from jax import lax
from jax.experimental import pallas as pl
from jax.experimental.pallas import tpu as pltpu
from jax.experimental.pallas import tpu_sc as plsc

N = 10000
E = 160000
VOCAB = 50000
L = 32
NFEAT = 256
NHID = 512
NCLASS = 128
B = 1000

NC = 2
NS = 16
NW = NC * NS

NPAD = 10240
NODES_PER_W = NPAD // NW
S1_NB = 2

EPW = E // NS
KB = 80
HALF = NPAD // 2
ACC4 = 6144
TRASH = 6000
XPAD = 1024

_mesh = plsc.VectorSubcoreMesh(
    core_axis_name="c", subcore_axis_name="s", num_cores=NC, num_subcores=NS)

_f32 = jnp.float32
_i32 = jnp.int32


def _zero_vmem_block(buf, rows):
    z = jnp.zeros((16,), _f32)
    for i in range(rows):
        for f in range(8):
            buf[i, pl.ds(f * 16, 16)] = z


@functools.partial(
    pl.kernel,
    out_type=jax.ShapeDtypeStruct((2 * NPAD, 128), _f32),
    mesh=_mesh,
    scratch_types=[
        pltpu.VMEM((S1_NB * L,), _i32),
        pltpu.VMEM((S1_NB * L, NFEAT), _f32),
        pltpu.VMEM((S1_NB, 128), _f32),
        pltpu.VMEM((S1_NB, 128), _f32),
        pltpu.SemaphoreType.DMA,
    ],
)
def _s1_embed(fi_hbm, emb_hbm, feats_hbm, idx_v, rows_v, olo_v, ohi_v, sem):
    c = lax.axis_index("c")
    s = lax.axis_index("s")
    wid = c * NS + s
    node0 = wid * NODES_PER_W

    def blk(i, carry):
        nb = node0 + i * S1_NB
        pltpu.sync_copy(fi_hbm.at[pl.ds(nb * L, S1_NB * L)], idx_v)
        pltpu.async_copy(emb_hbm.at[idx_v], rows_v, sem).wait()
        for n in range(S1_NB):
            for f in range(NFEAT // 16):
                acc = rows_v[n * L, pl.ds(f * 16, 16)]
                for t in range(1, L):
                    acc = acc + rows_v[n * L + t, pl.ds(f * 16, 16)]
                acc = acc * (1.0 / L)
                if f < 8:
                    olo_v[n, pl.ds(f * 16, 16)] = acc
                else:
                    ohi_v[n, pl.ds((f - 8) * 16, 16)] = acc
        pltpu.sync_copy(olo_v, feats_hbm.at[pl.ds(nb, S1_NB), :])
        pltpu.sync_copy(ohi_v, feats_hbm.at[pl.ds(NPAD + nb, S1_NB), :])
        return carry

    lax.fori_loop(0, NODES_PER_W // S1_NB, blk, 0)


@functools.partial(
    pl.kernel,
    out_type=jax.ShapeDtypeStruct((2 * NPAD, 128), _f32),
    mesh=_mesh,
    scratch_types=[
        pltpu.VMEM((KB,), _i32),
        pltpu.VMEM((KB,), _i32),
        pltpu.VMEM((KB,), _f32),
        pltpu.VMEM((KB, 128), _f32),
        pltpu.VMEM((64, 128), _f32),
        pltpu.VMEM_SHARED((NPAD, 128), _f32),
        pltpu.SemaphoreType.DMA,
    ],
)
def _s2_spmm1(src_hbm, dst_hbm, w_hbm, feats_hbm, t1_hbm,
              idx_v, dst_v, w_v, rows_v, zeros_v, acc_sh, sem):
    c = lax.axis_index("c")
    s = lax.axis_index("s")
    _zero_vmem_block(zeros_v, 64)
    for k in range(NPAD // NS // 64):
        pltpu.sync_copy(zeros_v, acc_sh.at[pl.ds(s * (NPAD // NS) + k * 64, 64), :])
    plsc.subcore_barrier()

    e0 = s * EPW

    def blk(i, carry):
        base = e0 + i * KB
        pltpu.sync_copy(src_hbm.at[pl.ds(base, KB)], idx_v)
        pltpu.sync_copy(dst_hbm.at[pl.ds(base, KB)], dst_v)
        pltpu.sync_copy(w_hbm.at[pl.ds(base, KB)], w_v)
        for j in range(KB // 16):
            idx_v[pl.ds(j * 16, 16)] = idx_v[pl.ds(j * 16, 16)] + c * NPAD
        pltpu.async_copy(feats_hbm.at[idx_v], rows_v, sem).wait()
        for j in range(KB // 16):
            wvec = w_v[pl.ds(j * 16, 16)]
            for t in range(16):
                e = j * 16 + t
                wv = jnp.full((16,), wvec[t], _f32)
                for f in range(8):
                    rows_v[e, pl.ds(f * 16, 16)] = rows_v[e, pl.ds(f * 16, 16)] * wv
        pltpu.sync_copy(rows_v, acc_sh.at[dst_v], add=True)
        return carry

    lax.fori_loop(0, EPW // KB, blk, 0)
    plsc.subcore_barrier()
    pltpu.sync_copy(acc_sh.at[pl.ds(s * (NPAD // NS), NPAD // NS), :],
                    t1_hbm.at[pl.ds(c * NPAD + s * (NPAD // NS), NPAD // NS), :])


def _tc_body(tlo_ref, thi_ref, w1_ref, b1_ref, w2_ref, z_ref):
    x = jnp.concatenate([tlo_ref[...], thi_ref[...]], axis=1)
    h = jnp.dot(x, w1_ref[...], preferred_element_type=_f32) + b1_ref[...]
    h = jnp.maximum(h, 0.0)
    z_ref[...] = jnp.dot(h, w2_ref[...], preferred_element_type=_f32)


_TC_BM = 512


def _tc_mlp(t1_cat, W1, b1, W2):
    nblk = NPAD // _TC_BM
    return pl.pallas_call(
        _tc_body,
        grid=(nblk,),
        in_specs=[
            pl.BlockSpec((_TC_BM, 128), lambda i: (i, 0)),
            pl.BlockSpec((_TC_BM, 128), lambda i: (i + NPAD // _TC_BM, 0)),
            pl.BlockSpec((NFEAT, NHID), lambda i: (0, 0)),
            pl.BlockSpec((1, NHID), lambda i: (0, 0)),
            pl.BlockSpec((NHID, NCLASS), lambda i: (0, 0)),
        ],
        out_specs=pl.BlockSpec((_TC_BM, NCLASS), lambda i: (i, 0)),
        out_shape=jax.ShapeDtypeStruct((NPAD, NCLASS), _f32),
    )(t1_cat, t1_cat, W1, b1.reshape(1, NHID), W2)


@functools.partial(
    pl.kernel,
    out_type=jax.ShapeDtypeStruct((2 * XPAD, NCLASS), _f32),
    mesh=_mesh,
    scratch_types=[
        pltpu.VMEM((KB,), _i32),
        pltpu.VMEM((KB,), _i32),
        pltpu.VMEM((KB,), _f32),
        pltpu.VMEM((KB, 128), _f32),
        pltpu.VMEM((64, 128), _f32),
        pltpu.VMEM((64,), _i32),
        pltpu.VMEM((64, 128), _f32),
        pltpu.VMEM((NCLASS,), _f32),
        pltpu.VMEM_SHARED((ACC4, 128), _f32),
        pltpu.SemaphoreType.DMA,
    ],
)
def _s4_spmm2(src_hbm, dst_hbm, w_hbm, z_hbm, xp_hbm, b2_hbm, outg_hbm,
              idx_v, dst_v, w_v, rows_v, zeros_v, xi_v, gout_v, b2_v, acc_sh, sem):
    c = lax.axis_index("c")
    s = lax.axis_index("s")
    _zero_vmem_block(zeros_v, 64)
    for k in range(ACC4 // NS // 64):
        pltpu.sync_copy(zeros_v, acc_sh.at[pl.ds(s * (ACC4 // NS) + k * 64, 64), :])
    pltpu.sync_copy(b2_hbm, b2_v)
    plsc.subcore_barrier()

    e0 = s * EPW

    def blk(i, carry):
        base = e0 + i * KB
        pltpu.sync_copy(src_hbm.at[pl.ds(base, KB)], idx_v)
        pltpu.sync_copy(dst_hbm.at[pl.ds(base, KB)], dst_v)
        pltpu.sync_copy(w_hbm.at[pl.ds(base, KB)], w_v)
        for j in range(KB // 16):
            d = dst_v[pl.ds(j * 16, 16)] - c * HALF
            ok = (d >= 0) & (d < HALF)
            dst_v[pl.ds(j * 16, 16)] = jnp.where(ok, d, TRASH)
        pltpu.async_copy(z_hbm.at[idx_v], rows_v, sem).wait()
        for j in range(KB // 16):
            wvec = w_v[pl.ds(j * 16, 16)]
            for t in range(16):
                e = j * 16 + t
                wv = jnp.full((16,), wvec[t], _f32)
                for f in range(8):
                    rows_v[e, pl.ds(f * 16, 16)] = rows_v[e, pl.ds(f * 16, 16)] * wv
        pltpu.sync_copy(rows_v, acc_sh.at[dst_v], add=True)
        return carry

    lax.fori_loop(0, EPW // KB, blk, 0)
    plsc.subcore_barrier()
    pltpu.sync_copy(xp_hbm.at[pl.ds(s * 64, 64)], xi_v)
    for j in range(4):
        d = xi_v[pl.ds(j * 16, 16)] - c * HALF
        xi_v[pl.ds(j * 16, 16)] = jnp.clip(d, 0, HALF - 1)
    pltpu.async_copy(acc_sh.at[xi_v], gout_v, sem).wait()
    for r in range(64):
        for f in range(8):
            gout_v[r, pl.ds(f * 16, 16)] = (
                gout_v[r, pl.ds(f * 16, 16)] + b2_v[pl.ds(f * 16, 16)])
    pltpu.sync_copy(gout_v, outg_hbm.at[pl.ds(c * XPAD + s * 64, 64), :])


def kernel(x_index, features_index, edge_index, edge_weight, embedding,
           W1, b1, W2, b2):
    fi_flat = jnp.pad(features_index, ((0, NPAD - N), (0, 0))).reshape(-1)
    src = edge_index[0]
    dst = edge_index[1]
    xp = jnp.pad(x_index, (0, XPAD - B))

    feats = _s1_embed(fi_flat, embedding)
    t1 = _s2_spmm1(src, dst, edge_weight, feats)
    z = _tc_mlp(t1, W1, b1, W2)
    outg = _s4_spmm2(src, dst, edge_weight, z, xp, b2)

    g = outg.reshape(2, XPAD, NCLASS)
    sel = (x_index < HALF)[:, None]
    return jnp.where(sel, g[0, :B], g[1, :B])


# speedup vs baseline: 3.5199x; 1.8790x over previous
import functools

import jax
import jax.numpy as jnp
from jax import lax
from jax.experimental import pallas as pl
from jax.experimental.pallas import tpu as pltpu
from jax.experimental.pallas import tpu_sc as plsc

N = 10000
E = 160000
VOCAB = 50000
L = 32
NFEAT = 256
NHID = 512
NCLASS = 128
B = 1000

NC = 2
NS = 16
NW = NC * NS

NPAD = 10240
NODES_PER_W = NPAD // NW
S1_NB = 2
S1_NBLK = NODES_PER_W // S1_NB

EPW = E // NS
KB = 80
NBLK = EPW // KB
HALF = NPAD // 2
ACC4 = 6144
TRASH = 6000
XPAD = 1024

_mesh = plsc.VectorSubcoreMesh(
    core_axis_name="c", subcore_axis_name="s", num_cores=NC, num_subcores=NS)

_f32 = jnp.float32
_i32 = jnp.int32


@functools.partial(
    pl.kernel,
    out_type=(jax.ShapeDtypeStruct((NPAD, 128), _f32),
              jax.ShapeDtypeStruct((NPAD, 128), _f32)),
    mesh=_mesh,
    scratch_types=[
        pltpu.VMEM((NODES_PER_W * L,), _i32),
        pltpu.VMEM((S1_NB * L, NFEAT), _f32),
        pltpu.VMEM((S1_NB * L, NFEAT), _f32),
        pltpu.VMEM((S1_NB, 128), _f32),
        pltpu.VMEM((S1_NB, 128), _f32),
        pltpu.VMEM((S1_NB, 128), _f32),
        pltpu.VMEM((S1_NB, 128), _f32),
        pltpu.SemaphoreType.DMA,
        pltpu.SemaphoreType.DMA,
        pltpu.SemaphoreType.DMA,
        pltpu.SemaphoreType.DMA,
    ],
)
def _s1_embed(fi_hbm, emb_hbm, flo_hbm, fhi_hbm,
              idxall_v, g0, g1, olo0, ohi0, olo1, ohi1,
              semg0, semg1, semo0, semo1):
    c = lax.axis_index("c")
    s = lax.axis_index("s")
    wid = c * NS + s
    node0 = wid * NODES_PER_W
    pltpu.sync_copy(fi_hbm.at[pl.ds(node0 * L, NODES_PER_W * L)], idxall_v)

    G = (g0, g1)
    OLO = (olo0, olo1)
    OHI = (ohi0, ohi1)
    SG = (semg0, semg1)
    SO = (semo0, semo1)

    def gidx(blk):
        return idxall_v.at[pl.ds(blk * S1_NB * L, S1_NB * L)]

    def rows_at(ref, blk):
        return ref.at[pl.ds(node0 + blk * S1_NB, S1_NB), :]

    pltpu.async_copy(emb_hbm.at[gidx(0)], g0, semg0)
    pltpu.async_copy(emb_hbm.at[gidx(1)], g1, semg1)

    def pair(g2, carry):
        for b in range(2):
            blk = g2 * 2 + b
            gbuf, olo, ohi, sg, so = G[b], OLO[b], OHI[b], SG[b], SO[b]
            pltpu.make_async_copy(emb_hbm.at[gidx(blk)], gbuf, sg).wait()

            @pl.when(g2 >= 1)
            def _():
                pltpu.make_async_copy(olo, rows_at(flo_hbm, blk - 2), so).wait()
                pltpu.make_async_copy(ohi, rows_at(fhi_hbm, blk - 2), so).wait()

            for n in range(S1_NB):
                for f in range(NFEAT // 16):
                    acc = gbuf[n * L, pl.ds(f * 16, 16)]
                    for t in range(1, L):
                        acc = acc + gbuf[n * L + t, pl.ds(f * 16, 16)]
                    acc = acc * (1.0 / L)
                    if f < 8:
                        olo[n, pl.ds(f * 16, 16)] = acc
                    else:
                        ohi[n, pl.ds((f - 8) * 16, 16)] = acc

            @pl.when(blk + 2 < S1_NBLK)
            def _():
                pltpu.async_copy(emb_hbm.at[gidx(blk + 2)], gbuf, sg)

            pltpu.async_copy(olo, rows_at(flo_hbm, blk), so)
            pltpu.async_copy(ohi, rows_at(fhi_hbm, blk), so)
        return carry

    lax.fori_loop(0, S1_NBLK // 2, pair, 0)
    pltpu.make_async_copy(olo0, rows_at(flo_hbm, S1_NBLK - 2), semo0).wait()
    pltpu.make_async_copy(ohi0, rows_at(fhi_hbm, S1_NBLK - 2), semo0).wait()
    pltpu.make_async_copy(olo1, rows_at(flo_hbm, S1_NBLK - 1), semo1).wait()
    pltpu.make_async_copy(ohi1, rows_at(fhi_hbm, S1_NBLK - 1), semo1).wait()


def _spmm_sweep(c, s, issue_gather, wait_gather, remap_dst, acc_sh, zero_rows,
                src_hbm, dst_hbm, w_hbm,
                i0, i1, w0, w1, ds0, ds1, dc0, dc1, g0, g1, sb0, sb1,
                semg0, semg1, sems0, sems1, semc0, semc1):
    z16 = jnp.zeros((16,), _f32)
    for i in range(64):
        for f in range(8):
            sb0[i, pl.ds(f * 16, 16)] = z16
    for k in range(zero_rows // 64):
        pltpu.sync_copy(sb0.at[pl.ds(0, 64), :],
                        acc_sh.at[pl.ds(s * zero_rows + k * 64, 64), :])
    plsc.subcore_barrier()

    G = (g0, g1)
    SB = (sb0, sb1)
    I = (i0, i1)
    W = (w0, w1)
    DS = (ds0, ds1)
    DC = (dc0, dc1)
    SG = (semg0, semg1)
    SS = (sems0, sems1)
    SC = (semc0, semc1)

    e0 = s * EPW

    def issue_chunks(blk, b):
        base = e0 + blk * KB
        pltpu.async_copy(src_hbm.at[pl.ds(base, KB)], I[b], SC[b])
        pltpu.async_copy(dst_hbm.at[pl.ds(base, KB)], DS[b], SC[b])
        pltpu.async_copy(w_hbm.at[pl.ds(base, KB)], W[b], SC[b])

    def wait_chunks(b):
        pltpu.make_async_copy(src_hbm.at[pl.ds(e0, KB)], I[b], SC[b]).wait()
        pltpu.make_async_copy(dst_hbm.at[pl.ds(e0, KB)], DS[b], SC[b]).wait()
        pltpu.make_async_copy(w_hbm.at[pl.ds(e0, KB)], W[b], SC[b]).wait()

    issue_chunks(0, 0)
    issue_chunks(1, 1)
    wait_chunks(0)
    issue_gather(i0, g0, semg0)

    def pair(g2, carry):
        for b in range(2):
            blk = g2 * 2 + b

            @pl.when(blk < NBLK)
            def _():
                gbuf, sbuf, dstage, dscat = G[b], SB[b], DS[b], DC[b]
                sg, ss = SG[b], SS[b]
                wait_gather(I[b], gbuf, sg)

                @pl.when(blk >= 2)
                def _():
                    pltpu.make_async_copy(sbuf, acc_sh.at[dscat], ss).wait()

                for j in range(KB // 16):
                    wvec = W[b][pl.ds(j * 16, 16)]
                    for t in range(16):
                        e = j * 16 + t
                        wv = jnp.full((16,), wvec[t], _f32)
                        for f in range(8):
                            sbuf[e, pl.ds(f * 16, 16)] = (
                                gbuf[e, pl.ds(f * 16, 16)] * wv)

                for j in range(KB // 16):
                    dscat[pl.ds(j * 16, 16)] = remap_dst(
                        dstage[pl.ds(j * 16, 16)])

                pltpu.async_copy(sbuf, acc_sh.at[dscat], ss, add=True)

                @pl.when(blk + 2 < NBLK)
                def _():
                    issue_chunks(blk + 2, b)

                @pl.when(blk + 1 < NBLK)
                def _():
                    wait_chunks(1 - b)
                    issue_gather(I[1 - b], G[1 - b], SG[1 - b])
        return carry

    lax.fori_loop(0, (NBLK + 1) // 2, pair, 0)
    pltpu.make_async_copy(sb1, acc_sh.at[dc1], sems1).wait()
    pltpu.make_async_copy(sb0, acc_sh.at[dc0], sems0).wait()
    plsc.subcore_barrier()


def _spmm_scratch(rows):
    return [
        pltpu.VMEM((KB,), _i32),
        pltpu.VMEM((KB,), _i32),
        pltpu.VMEM((KB,), _f32),
        pltpu.VMEM((KB,), _f32),
        pltpu.VMEM((KB,), _i32),
        pltpu.VMEM((KB,), _i32),
        pltpu.VMEM((KB,), _i32),
        pltpu.VMEM((KB,), _i32),
        pltpu.VMEM((KB, 128), _f32),
        pltpu.VMEM((KB, 128), _f32),
        pltpu.VMEM((KB, 128), _f32),
        pltpu.VMEM((KB, 128), _f32),
        pltpu.VMEM_SHARED((rows, 128), _f32),
        pltpu.SemaphoreType.DMA,
        pltpu.SemaphoreType.DMA,
        pltpu.SemaphoreType.DMA,
        pltpu.SemaphoreType.DMA,
        pltpu.SemaphoreType.DMA,
        pltpu.SemaphoreType.DMA,
    ]


@functools.partial(
    pl.kernel,
    out_type=jax.ShapeDtypeStruct((2 * NPAD, 128), _f32),
    mesh=_mesh,
    scratch_types=_spmm_scratch(NPAD),
)
def _s2_spmm1(src_hbm, dst_hbm, w_hbm, flo_hbm, fhi_hbm, t1_hbm,
              i0, i1, w0, w1, ds0, ds1, dc0, dc1, g0, g1, sb0, sb1,
              acc_sh, semg0, semg1, sems0, sems1, semc0, semc1):
    c = lax.axis_index("c")
    s = lax.axis_index("s")

    def issue_gather(ibuf, gbuf, sg):
        @pl.when(c == 0)
        def _():
            pltpu.async_copy(flo_hbm.at[ibuf], gbuf, sg)

        @pl.when(c == 1)
        def _():
            pltpu.async_copy(fhi_hbm.at[ibuf], gbuf, sg)

    def wait_gather(ibuf, gbuf, sg):
        pltpu.make_async_copy(flo_hbm.at[ibuf], gbuf, sg).wait()

    _spmm_sweep(c, s, issue_gather, wait_gather, lambda d: d, acc_sh, NPAD // NS,
                src_hbm, dst_hbm, w_hbm,
                i0, i1, w0, w1, ds0, ds1, dc0, dc1, g0, g1, sb0, sb1,
                semg0, semg1, sems0, sems1, semc0, semc1)
    rpw = NPAD // NS
    pltpu.sync_copy(acc_sh.at[pl.ds(s * rpw, rpw), :],
                    t1_hbm.at[pl.ds(c * NPAD + s * rpw, rpw), :])


def _tc_body(tlo_ref, thi_ref, w1_ref, b1_ref, w2_ref, z_ref):
    x = jnp.concatenate([tlo_ref[...], thi_ref[...]], axis=1)
    h = jnp.dot(x, w1_ref[...], preferred_element_type=_f32) + b1_ref[...]
    h = jnp.maximum(h, 0.0)
    z_ref[...] = jnp.dot(h, w2_ref[...], preferred_element_type=_f32)


_TC_BM = 512


def _tc_mlp(t1_cat, W1, b1, W2):
    nblk = NPAD // _TC_BM
    return pl.pallas_call(
        _tc_body,
        grid=(nblk,),
        in_specs=[
            pl.BlockSpec((_TC_BM, 128), lambda i: (i, 0)),
            pl.BlockSpec((_TC_BM, 128), lambda i: (i + NPAD // _TC_BM, 0)),
            pl.BlockSpec((NFEAT, NHID), lambda i: (0, 0)),
            pl.BlockSpec((1, NHID), lambda i: (0, 0)),
            pl.BlockSpec((NHID, NCLASS), lambda i: (0, 0)),
        ],
        out_specs=pl.BlockSpec((_TC_BM, NCLASS), lambda i: (i, 0)),
        out_shape=jax.ShapeDtypeStruct((NPAD, NCLASS), _f32),
    )(t1_cat, t1_cat, W1, b1.reshape(1, NHID), W2)


@functools.partial(
    pl.kernel,
    out_type=jax.ShapeDtypeStruct((2 * XPAD, NCLASS), _f32),
    mesh=_mesh,
    scratch_types=_spmm_scratch(ACC4) + [
        pltpu.VMEM((64,), _i32),
        pltpu.VMEM((64, NCLASS), _f32),
        pltpu.VMEM((NCLASS,), _f32),
    ],
)
def _s4_spmm2(src_hbm, dst_hbm, w_hbm, z_hbm, xp_hbm, b2_hbm, outg_hbm,
              i0, i1, w0, w1, ds0, ds1, dc0, dc1, g0, g1, sb0, sb1,
              acc_sh, semg0, semg1, sems0, sems1, semc0, semc1,
              xi_v, gout_v, b2_v):
    c = lax.axis_index("c")
    s = lax.axis_index("s")
    pltpu.sync_copy(b2_hbm, b2_v)

    def issue_gather(ibuf, gbuf, sg):
        pltpu.async_copy(z_hbm.at[ibuf], gbuf, sg)

    def wait_gather(ibuf, gbuf, sg):
        pltpu.make_async_copy(z_hbm.at[ibuf], gbuf, sg).wait()

    def remap_dst(d):
        v = d - c * HALF
        ok = (v >= 0) & (v < HALF)
        return jnp.where(ok, v, TRASH)

    _spmm_sweep(c, s, issue_gather, wait_gather, remap_dst, acc_sh, ACC4 // NS,
                src_hbm, dst_hbm, w_hbm,
                i0, i1, w0, w1, ds0, ds1, dc0, dc1, g0, g1, sb0, sb1,
                semg0, semg1, sems0, sems1, semc0, semc1)

    pltpu.sync_copy(xp_hbm.at[pl.ds(s * 64, 64)], xi_v)
    for j in range(4):
        v = xi_v[pl.ds(j * 16, 16)] - c * HALF
        xi_v[pl.ds(j * 16, 16)] = jnp.clip(v, 0, HALF - 1)
    pltpu.async_copy(acc_sh.at[xi_v], gout_v, semg0).wait()
    for r in range(64):
        for f in range(8):
            gout_v[r, pl.ds(f * 16, 16)] = (
                gout_v[r, pl.ds(f * 16, 16)] + b2_v[pl.ds(f * 16, 16)])
    pltpu.sync_copy(gout_v, outg_hbm.at[pl.ds(c * XPAD + s * 64, 64), :])


def kernel(x_index, features_index, edge_index, edge_weight, embedding,
           W1, b1, W2, b2):
    fi_flat = jnp.pad(features_index, ((0, NPAD - N), (0, 0))).reshape(-1)
    src = edge_index[0]
    dst = edge_index[1]
    xp = jnp.pad(x_index, (0, XPAD - B))

    flo, fhi = _s1_embed(fi_flat, embedding)
    t1 = _s2_spmm1(src, dst, edge_weight, flo, fhi)
    z = _tc_mlp(t1, W1, b1, W2)
    outg = _s4_spmm2(src, dst, edge_weight, z, xp, b2)

    g = outg.reshape(2, XPAD, NCLASS)
    sel = (x_index < HALF)[:, None]
    return jnp.where(sel, g[0, :B], g[1, :B])
